# Initial kernel scaffold; baseline (speedup 1.0000x reference)
import jax
import jax.numpy as jnp
from jax.experimental import pallas as pl


def kernel(ent_emb, rel_emb0, rel_emb1, We0, Wn0, Wc0, We1, Wn1, Wc1, pred_rel, h_id, r_id, edge_index, rel_id):
    raise NotImplementedError("write your pallas kernel here")


# speedup vs baseline: 1.0570x; 1.0570x over previous
import functools
import jax
import jax.numpy as jnp
from jax import lax
from jax.experimental import pallas as pl
from jax.experimental.pallas import tpu as pltpu, tpu_sc as plsc

N_NODES = 10000
N_RELROWS = 1000
E_EDGES = 160000
DIM = 256
BATCH = 512

NC = 2
NS = 16
NWORK = NC * NS
EB = 128
NBLK = E_EDGES // EB
QD = 32
NQ = DIM // QD
NROUND = NQ // NC
ACC_ROWS = 3 * N_NODES
ACC_PAD = 30720
STRIPE = ACC_PAD // NS
RB = 128
NWB = STRIPE // RB

_mesh = plsc.VectorSubcoreMesh(
    core_axis_name="c", subcore_axis_name="s", num_cores=NC, num_subcores=NS)
_sc_params = pltpu.CompilerParams(
    use_tc_tiling_on_sc=False, needs_layout_passes=False)


def _iota16():
  return lax.iota(jnp.int32, 16)


def _full16(v):
  return jnp.full((16,), v, jnp.int32)


def _scores_body(ent_hbm, rel_hbm, src_hbm, dst_hbm, rid_hbm, ex_hbm,
                 srcv, dstv, ridv, u_rows, v_rows, e_rows, exbuf, sem):
  c = lax.axis_index("c")
  s = lax.axis_index("s")
  w = s * NC + c
  nb = (NBLK - w + NWORK - 1) // NWORK

  def blk_body(t, carry):
    blk = w + t * NWORK
    base = blk * EB
    pltpu.sync_copy(src_hbm.at[pl.ds(base, EB)], srcv)
    pltpu.sync_copy(dst_hbm.at[pl.ds(base, EB)], dstv)
    pltpu.sync_copy(rid_hbm.at[pl.ds(base, EB)], ridv)
    pltpu.async_copy(ent_hbm.at[srcv], u_rows, sem).wait()
    pltpu.async_copy(ent_hbm.at[dstv], v_rows, sem).wait()
    pltpu.async_copy(rel_hbm.at[ridv], e_rows, sem).wait()
    for g in range(EB // 16):
      rows16 = _iota16() + g * 16
      z = jnp.zeros((16,), jnp.float32)

      def col(d, acc):
        a1, a2, a3 = acc
        dc = _full16(d)
        u = plsc.load_gather(u_rows, [rows16, dc])
        v = plsc.load_gather(v_rows, [rows16, dc])
        e = plsc.load_gather(e_rows, [rows16, dc])
        ev = e * v
        return (a1 + ev, a2 + u * v, a3 + u * ev)

      a1, a2, a3 = lax.fori_loop(0, DIM, col, (z, z, z))
      sl = pl.ds(g * 16, 16)
      exbuf[0, sl] = jnp.exp(a1)
      exbuf[1, sl] = jnp.exp(a2)
      exbuf[2, sl] = jnp.exp(a3)
    for j in range(3):
      pltpu.sync_copy(exbuf.at[j], ex_hbm.at[j, blk])
    return carry

  lax.fori_loop(0, nb, blk_body, 0)


_scores_kernel = pl.kernel(
    _scores_body,
    out_type=jax.ShapeDtypeStruct((3, NBLK, EB), jnp.float32),
    mesh=_mesh,
    scratch_types=[
        pltpu.VMEM((EB,), jnp.int32),
        pltpu.VMEM((EB,), jnp.int32),
        pltpu.VMEM((EB,), jnp.int32),
        pltpu.VMEM((EB, DIM), jnp.float32),
        pltpu.VMEM((EB, DIM), jnp.float32),
        pltpu.VMEM((EB, DIM), jnp.float32),
        pltpu.VMEM((3, EB), jnp.float32),
        pltpu.SemaphoreType.DMA,
    ],
    compiler_params=_sc_params,
)


def _agg_body(rnd, entq_hbm, relq_hbm, src_hbm, dst_hbm, rid_hbm, ex_hbm,
              zacc_hbm, zden_hbm, out_hbm,
              srcv, dstv, ridv, giu, gie, sidx, exd, u_q, e_q, msg,
              rbuf, dbuf, acc_sh, den_sh, sem):
  c = lax.axis_index("c")
  s = lax.axis_index("s")
  q = rnd * NC + c

  @pl.when(s == 0)
  def _init():
    pltpu.sync_copy(zacc_hbm, acc_sh)
    pltpu.sync_copy(zden_hbm, den_sh)

  plsc.subcore_barrier()

  nb = (NBLK - s + NS - 1) // NS
  qent = q * N_NODES
  qrel = q * N_RELROWS

  def blk_body(t, carry):
    blk = s + t * NS
    base = blk * EB
    pltpu.sync_copy(src_hbm.at[pl.ds(base, EB)], srcv)
    pltpu.sync_copy(dst_hbm.at[pl.ds(base, EB)], dstv)
    pltpu.sync_copy(rid_hbm.at[pl.ds(base, EB)], ridv)
    for j in range(3):
      pltpu.sync_copy(ex_hbm.at[j, blk], exd.at[pl.ds(j * EB, EB)])
    for g in range(EB // 16):
      sl = pl.ds(g * 16, 16)
      sv = srcv[sl]
      dv = dstv[sl]
      rv = ridv[sl]
      giu[sl] = sv + qent
      gie[sl] = rv + qrel
      sidx[sl] = dv
      sidx[pl.ds(EB + g * 16, 16)] = dv + N_NODES
      sidx[pl.ds(2 * EB + g * 16, 16)] = dv + 2 * N_NODES
    pltpu.async_copy(entq_hbm.at[giu], u_q, sem).wait()
    pltpu.async_copy(relq_hbm.at[gie], e_q, sem).wait()
    for g in range(EB // 16):
      rows16 = _iota16() + g * 16
      ex1 = exd[pl.ds(g * 16, 16)]
      ex2 = exd[pl.ds(EB + g * 16, 16)]
      ex3 = exd[pl.ds(2 * EB + g * 16, 16)]

      def col(d, carry2):
        dc = _full16(d)
        u = plsc.load_gather(u_q, [rows16, dc])
        e = plsc.load_gather(e_q, [rows16, dc])
        plsc.store_scatter(msg, [rows16, dc], e * ex1)
        plsc.store_scatter(msg, [rows16 + EB, dc], u * ex2)
        plsc.store_scatter(msg, [rows16 + 2 * EB, dc], u * e * ex3)
        return carry2

      lax.fori_loop(0, QD, col, 0)
    pltpu.sync_copy(msg, acc_sh.at[sidx], add=True)
    pltpu.sync_copy(exd, den_sh.at[sidx], add=True)
    return carry

  lax.fori_loop(0, nb, blk_body, 0)
  plsc.subcore_barrier()

  def wb_body(k, carry):
    r0 = s * STRIPE + k * RB
    pltpu.sync_copy(acc_sh.at[pl.ds(r0, RB)], rbuf)
    pltpu.sync_copy(den_sh.at[pl.ds(r0, RB)], dbuf)

    def row(i, carry2):
      dscal = plsc.load_gather(dbuf, [_full16(i)])
      rec = 1.0 / (dscal + 1e-16)
      for cc in range(QD // 16):
        sl = pl.ds(cc * 16, 16)
        rbuf[i, sl] = rbuf[i, sl] * rec
      return carry2

    lax.fori_loop(0, RB, row, 0)
    pltpu.sync_copy(rbuf, out_hbm.at[c, pl.ds(r0, RB)])
    return carry

  lax.fori_loop(0, NWB, wb_body, 0)


def _make_agg_kernel(rnd):
  return pl.kernel(
      functools.partial(_agg_body, rnd),
      out_type=jax.ShapeDtypeStruct((NC, ACC_PAD, QD), jnp.float32),
      mesh=_mesh,
      scratch_types=[
          pltpu.VMEM((EB,), jnp.int32),
          pltpu.VMEM((EB,), jnp.int32),
          pltpu.VMEM((EB,), jnp.int32),
          pltpu.VMEM((EB,), jnp.int32),
          pltpu.VMEM((EB,), jnp.int32),
          pltpu.VMEM((3 * EB,), jnp.int32),
          pltpu.VMEM((3 * EB,), jnp.float32),
          pltpu.VMEM((EB, QD), jnp.float32),
          pltpu.VMEM((EB, QD), jnp.float32),
          pltpu.VMEM((3 * EB, QD), jnp.float32),
          pltpu.VMEM((RB, QD), jnp.float32),
          pltpu.VMEM((RB,), jnp.float32),
          pltpu.VMEM_SHARED((ACC_PAD, QD), jnp.float32),
          pltpu.VMEM_SHARED((ACC_PAD,), jnp.float32),
          pltpu.SemaphoreType.DMA,
      ],
      compiler_params=_sc_params,
  )


_agg_kernels = [_make_agg_kernel(r) for r in range(NROUND)]


def _combine_body(ent_ref, m_ref, w_ref, out_ref):
  acc = ent_ref[...]
  for b in range(3):
    acc = acc + jnp.tanh(
        jnp.dot(m_ref[b], w_ref[b], preferred_element_type=jnp.float32))
  out_ref[...] = acc


_ROWBLK = 1000


def _combine(ent, m, w):
  return pl.pallas_call(
      _combine_body,
      grid=(N_NODES // _ROWBLK,),
      in_specs=[
          pl.BlockSpec((_ROWBLK, DIM), lambda i: (i, 0)),
          pl.BlockSpec((3, _ROWBLK, DIM), lambda i: (0, i, 0)),
          pl.BlockSpec((3, DIM, DIM), lambda i: (0, 0, 0)),
      ],
      out_specs=pl.BlockSpec((_ROWBLK, DIM), lambda i: (i, 0)),
      out_shape=jax.ShapeDtypeStruct((N_NODES, DIM), jnp.float32),
  )(ent, m, w)


def _headq_body(ent_hbm, prel_hbm, h_hbm, r_hbm, out_hbm,
                hv, rv, hrows, rrows, obuf, sem):
  c = lax.axis_index("c")
  s = lax.axis_index("s")
  w = s * NC + c
  base = w * 16
  pltpu.sync_copy(h_hbm.at[pl.ds(base, 16)], hv)
  pltpu.sync_copy(r_hbm.at[pl.ds(base, 16)], rv)
  pltpu.async_copy(ent_hbm.at[hv], hrows, sem).wait()
  pltpu.async_copy(prel_hbm.at[rv], rrows, sem).wait()

  def row(i, carry):
    for cc in range(DIM // 16):
      sl = pl.ds(cc * 16, 16)
      obuf[i, sl] = hrows[i, sl] * rrows[i, sl]
    return carry

  lax.fori_loop(0, 16, row, 0)
  pltpu.sync_copy(obuf, out_hbm.at[pl.ds(base, 16)])


_headq_kernel = pl.kernel(
    _headq_body,
    out_type=jax.ShapeDtypeStruct((BATCH, DIM), jnp.float32),
    mesh=_mesh,
    scratch_types=[
        pltpu.VMEM((16,), jnp.int32),
        pltpu.VMEM((16,), jnp.int32),
        pltpu.VMEM((16, DIM), jnp.float32),
        pltpu.VMEM((16, DIM), jnp.float32),
        pltpu.VMEM((16, DIM), jnp.float32),
        pltpu.SemaphoreType.DMA,
    ],
    compiler_params=_sc_params,
)


def _score_body(hq_ref, ent_ref, out_ref):
  out_ref[...] = lax.dot_general(
      hq_ref[...], ent_ref[...],
      dimension_numbers=(((1,), (1,)), ((), ())),
      preferred_element_type=jnp.float32)


def _score(hq, ent):
  return pl.pallas_call(
      _score_body,
      out_shape=jax.ShapeDtypeStruct((BATCH, N_NODES), jnp.float32),
  )(hq, ent)


def _slices(table):
  r = table.shape[0]
  return jnp.transpose(table.reshape(r, NQ, QD), (1, 0, 2)).reshape(NQ * r, QD)


def _layer(ent, rel, wstack, src, dst, rid, zacc, zden):
  ex = _scores_kernel(ent, rel, src, dst, rid)
  entq = _slices(ent)
  relq = _slices(rel)
  outs = [k(entq, relq, src, dst, rid, ex, zacc, zden) for k in _agg_kernels]
  m = jnp.concatenate([o[c] for o in outs for c in range(NC)], axis=-1)
  m = m[:ACC_ROWS].reshape(3, N_NODES, DIM)
  return _combine(ent, m, wstack)


def kernel(ent_emb, rel_emb0, rel_emb1, We0, Wn0, Wc0, We1, Wn1, Wc1,
           pred_rel, h_id, r_id, edge_index, rel_id):
  src = edge_index[0].astype(jnp.int32)
  dst = edge_index[1].astype(jnp.int32)
  rid = rel_id.astype(jnp.int32)
  zacc = jnp.zeros((ACC_PAD, QD), jnp.float32)
  zden = jnp.zeros((ACC_PAD,), jnp.float32)
  w0 = jnp.stack([We0, Wn0, Wc0])
  w1 = jnp.stack([We1, Wn1, Wc1])
  ent1 = _layer(ent_emb, rel_emb0, w0, src, dst, rid, zacc, zden)
  ent2 = _layer(ent1, rel_emb1, w1, src, dst, rid, zacc, zden)
  hq = _headq_kernel(ent2, pred_rel, h_id.astype(jnp.int32),
                     r_id.astype(jnp.int32))
  return _score(hq, ent2)


# speedup vs baseline: 3.1041x; 2.9366x over previous
import functools
import jax
import jax.numpy as jnp
from jax import lax
from jax.experimental import pallas as pl
from jax.experimental.pallas import tpu as pltpu, tpu_sc as plsc

N_NODES = 10000
N_RELROWS = 1000
E_EDGES = 160000
DIM = 256
BATCH = 512

NC = 2
NS = 16
NWORK = NC * NS
EB = 128
NBLK = E_EDGES // EB
QD = 32
NQ = DIM // QD
NROUND = NQ // NC
ACC_ROWS = 3 * N_NODES
ACC_PAD = 30720
STRIPE = ACC_PAD // NS
RB = 128
NWB = STRIPE // RB

_mesh = plsc.VectorSubcoreMesh(
    core_axis_name="c", subcore_axis_name="s", num_cores=NC, num_subcores=NS)
_sc_params = pltpu.CompilerParams(
    use_tc_tiling_on_sc=False, needs_layout_passes=False)


def _iota16():
  return lax.iota(jnp.int32, 16)


def _full16(v):
  return jnp.full((16,), v, jnp.int32)


def _scores_body(ent_hbm, rel_hbm, src_hbm, dst_hbm, rid_hbm, zden_hbm,
                 ex_hbm, den_hbm,
                 srcv, dstv, ridv, u_rows, v_rows, e_rows, pbuf, exd, sidx,
                 den_sh, sem):
  c = lax.axis_index("c")
  s = lax.axis_index("s")
  w = s * NC + c
  nb = (NBLK - w + NWORK - 1) // NWORK

  @pl.when(s == 0)
  def _init():
    pltpu.sync_copy(zden_hbm, den_sh)

  plsc.subcore_barrier()

  def blk_body(t, carry):
    blk = w + t * NWORK
    base = blk * EB
    pltpu.sync_copy(src_hbm.at[pl.ds(base, EB)], srcv)
    pltpu.sync_copy(dst_hbm.at[pl.ds(base, EB)], dstv)
    pltpu.sync_copy(rid_hbm.at[pl.ds(base, EB)], ridv)
    pltpu.async_copy(ent_hbm.at[srcv], u_rows, sem).wait()
    pltpu.async_copy(ent_hbm.at[dstv], v_rows, sem).wait()
    pltpu.async_copy(rel_hbm.at[ridv], e_rows, sem).wait()

    def edge(i, carry2):
      z = jnp.zeros((16,), jnp.float32)
      a1 = z
      a2 = z
      a3 = z
      for cc in range(DIM // 16):
        sl = pl.ds(cc * 16, 16)
        u = u_rows[i, sl]
        v = v_rows[i, sl]
        e = e_rows[i, sl]
        ev = e * v
        a1 = a1 + ev
        a2 = a2 + u * v
        a3 = a3 + u * ev
      pbuf[i] = plsc.cumsum(a1)
      pbuf[EB + i] = plsc.cumsum(a2)
      pbuf[2 * EB + i] = plsc.cumsum(a3)
      return carry2

    lax.fori_loop(0, EB, edge, 0)
    f15 = _full16(15)
    for g in range(EB // 16):
      rows16 = _iota16() + g * 16
      dv = dstv[pl.ds(g * 16, 16)]
      for j in range(3):
        ssum = plsc.load_gather(pbuf, [j * EB + rows16, f15])
        exd[pl.ds(j * EB + g * 16, 16)] = jnp.exp(ssum)
        sidx[pl.ds(j * EB + g * 16, 16)] = dv + j * N_NODES
    pltpu.sync_copy(exd, den_sh.at[sidx], add=True)
    pltpu.sync_copy(exd, ex_hbm.at[blk])
    return carry

  lax.fori_loop(0, nb, blk_body, 0)
  plsc.subcore_barrier()
  pltpu.sync_copy(den_sh.at[pl.ds(s * STRIPE, STRIPE)],
                  den_hbm.at[c, pl.ds(s * STRIPE, STRIPE)])


_scores_kernel = pl.kernel(
    _scores_body,
    out_type=(jax.ShapeDtypeStruct((NBLK, 3 * EB), jnp.float32),
              jax.ShapeDtypeStruct((NC, ACC_PAD), jnp.float32)),
    mesh=_mesh,
    scratch_types=[
        pltpu.VMEM((EB,), jnp.int32),
        pltpu.VMEM((EB,), jnp.int32),
        pltpu.VMEM((EB,), jnp.int32),
        pltpu.VMEM((EB, DIM), jnp.float32),
        pltpu.VMEM((EB, DIM), jnp.float32),
        pltpu.VMEM((EB, DIM), jnp.float32),
        pltpu.VMEM((3 * EB, 16), jnp.float32),
        pltpu.VMEM((3 * EB,), jnp.float32),
        pltpu.VMEM((3 * EB,), jnp.int32),
        pltpu.VMEM_SHARED((ACC_PAD,), jnp.float32),
        pltpu.SemaphoreType.DMA,
    ],
    compiler_params=_sc_params,
)


def _agg_body(rnd, entq_hbm, relq_hbm, src_hbm, dst_hbm, rid_hbm, ex_hbm,
              zacc_hbm, out_hbm,
              srcv, dstv, ridv, giu, gie, sidx, exd, u_q, e_q, msg,
              acc_sh, sem):
  c = lax.axis_index("c")
  s = lax.axis_index("s")
  q = rnd * NC + c

  @pl.when(s == 0)
  def _init():
    pltpu.sync_copy(zacc_hbm, acc_sh)

  plsc.subcore_barrier()

  nb = (NBLK - s + NS - 1) // NS
  qent = q * N_NODES
  qrel = q * N_RELROWS

  def blk_body(t, carry):
    blk = s + t * NS
    base = blk * EB
    pltpu.sync_copy(src_hbm.at[pl.ds(base, EB)], srcv)
    pltpu.sync_copy(dst_hbm.at[pl.ds(base, EB)], dstv)
    pltpu.sync_copy(rid_hbm.at[pl.ds(base, EB)], ridv)
    pltpu.sync_copy(ex_hbm.at[blk], exd)
    for g in range(EB // 16):
      sl = pl.ds(g * 16, 16)
      sv = srcv[sl]
      dv = dstv[sl]
      rv = ridv[sl]
      giu[sl] = sv + qent
      gie[sl] = rv + qrel
      sidx[sl] = dv
      sidx[pl.ds(EB + g * 16, 16)] = dv + N_NODES
      sidx[pl.ds(2 * EB + g * 16, 16)] = dv + 2 * N_NODES
    pltpu.async_copy(entq_hbm.at[giu], u_q, sem).wait()
    pltpu.async_copy(relq_hbm.at[gie], e_q, sem).wait()

    def edge(i, carry2):
      e1 = plsc.load_gather(exd, [_full16(i)])
      e2 = plsc.load_gather(exd, [_full16(EB + i)])
      e3 = plsc.load_gather(exd, [_full16(2 * EB + i)])
      for cc in range(QD // 16):
        sl = pl.ds(cc * 16, 16)
        u = u_q[i, sl]
        e = e_q[i, sl]
        msg[i, sl] = e * e1
        msg[EB + i, sl] = u * e2
        msg[2 * EB + i, sl] = u * e * e3
      return carry2

    lax.fori_loop(0, EB, edge, 0)
    pltpu.sync_copy(msg, acc_sh.at[sidx], add=True)
    return carry

  lax.fori_loop(0, nb, blk_body, 0)
  plsc.subcore_barrier()
  pltpu.sync_copy(acc_sh.at[pl.ds(s * STRIPE, STRIPE)],
                  out_hbm.at[c, pl.ds(s * STRIPE, STRIPE)])


def _make_agg_kernel(rnd):
  return pl.kernel(
      functools.partial(_agg_body, rnd),
      out_type=jax.ShapeDtypeStruct((NC, ACC_PAD, QD), jnp.float32),
      mesh=_mesh,
      scratch_types=[
          pltpu.VMEM((EB,), jnp.int32),
          pltpu.VMEM((EB,), jnp.int32),
          pltpu.VMEM((EB,), jnp.int32),
          pltpu.VMEM((EB,), jnp.int32),
          pltpu.VMEM((EB,), jnp.int32),
          pltpu.VMEM((3 * EB,), jnp.int32),
          pltpu.VMEM((3 * EB,), jnp.float32),
          pltpu.VMEM((EB, QD), jnp.float32),
          pltpu.VMEM((EB, QD), jnp.float32),
          pltpu.VMEM((3 * EB, QD), jnp.float32),
          pltpu.VMEM_SHARED((ACC_PAD, QD), jnp.float32),
          pltpu.SemaphoreType.DMA,
      ],
      compiler_params=_sc_params,
  )


_agg_kernels = [_make_agg_kernel(r) for r in range(NROUND)]


def _combine_body(ent_ref, m_ref, den_ref, w_ref, out_ref):
  den = den_ref[...]
  den = den[:, :, 0] + den[:, :, 1] + 1e-16
  acc = ent_ref[...]
  for b in range(3):
    mb = m_ref[b] * (1.0 / den[:, b])[:, None]
    acc = acc + jnp.tanh(
        jnp.dot(mb, w_ref[b], preferred_element_type=jnp.float32))
  out_ref[...] = acc


_ROWBLK = 1000


def _combine(ent, m, den_t, w):
  return pl.pallas_call(
      _combine_body,
      grid=(N_NODES // _ROWBLK,),
      in_specs=[
          pl.BlockSpec((_ROWBLK, DIM), lambda i: (i, 0)),
          pl.BlockSpec((3, _ROWBLK, DIM), lambda i: (0, i, 0)),
          pl.BlockSpec((_ROWBLK, 3, 2), lambda i: (i, 0, 0)),
          pl.BlockSpec((3, DIM, DIM), lambda i: (0, 0, 0)),
      ],
      out_specs=pl.BlockSpec((_ROWBLK, DIM), lambda i: (i, 0)),
      out_shape=jax.ShapeDtypeStruct((N_NODES, DIM), jnp.float32),
  )(ent, m, den_t, w)


def _headq_body(ent_hbm, prel_hbm, h_hbm, r_hbm, out_hbm,
                hv, rv, hrows, rrows, obuf, sem):
  c = lax.axis_index("c")
  s = lax.axis_index("s")
  w = s * NC + c
  base = w * 16
  pltpu.sync_copy(h_hbm.at[pl.ds(base, 16)], hv)
  pltpu.sync_copy(r_hbm.at[pl.ds(base, 16)], rv)
  pltpu.async_copy(ent_hbm.at[hv], hrows, sem).wait()
  pltpu.async_copy(prel_hbm.at[rv], rrows, sem).wait()

  def row(i, carry):
    for cc in range(DIM // 16):
      sl = pl.ds(cc * 16, 16)
      obuf[i, sl] = hrows[i, sl] * rrows[i, sl]
    return carry

  lax.fori_loop(0, 16, row, 0)
  pltpu.sync_copy(obuf, out_hbm.at[pl.ds(base, 16)])


_headq_kernel = pl.kernel(
    _headq_body,
    out_type=jax.ShapeDtypeStruct((BATCH, DIM), jnp.float32),
    mesh=_mesh,
    scratch_types=[
        pltpu.VMEM((16,), jnp.int32),
        pltpu.VMEM((16,), jnp.int32),
        pltpu.VMEM((16, DIM), jnp.float32),
        pltpu.VMEM((16, DIM), jnp.float32),
        pltpu.VMEM((16, DIM), jnp.float32),
        pltpu.SemaphoreType.DMA,
    ],
    compiler_params=_sc_params,
)


def _score_body(hq_ref, ent_ref, out_ref):
  out_ref[...] = lax.dot_general(
      hq_ref[...], ent_ref[...],
      dimension_numbers=(((1,), (1,)), ((), ())),
      preferred_element_type=jnp.float32)


def _score(hq, ent):
  return pl.pallas_call(
      _score_body,
      out_shape=jax.ShapeDtypeStruct((BATCH, N_NODES), jnp.float32),
  )(hq, ent)


def _slices(table):
  r = table.shape[0]
  return jnp.transpose(table.reshape(r, NQ, QD), (1, 0, 2)).reshape(NQ * r, QD)


def _layer(ent, rel, wstack, src, dst, rid, zacc, zden):
  ex, den = _scores_kernel(ent, rel, src, dst, rid, zden)
  entq = _slices(ent)
  relq = _slices(rel)
  outs = [k(entq, relq, src, dst, rid, ex, zacc) for k in _agg_kernels]
  m = jnp.concatenate([o[c] for o in outs for c in range(NC)], axis=-1)
  m = m[:ACC_ROWS].reshape(3, N_NODES, DIM)
  den_t = jnp.transpose(den[:, :ACC_ROWS].reshape(NC, 3, N_NODES), (2, 1, 0))
  return _combine(ent, m, den_t, wstack)


def kernel(ent_emb, rel_emb0, rel_emb1, We0, Wn0, Wc0, We1, Wn1, Wc1,
           pred_rel, h_id, r_id, edge_index, rel_id):
  src = edge_index[0].astype(jnp.int32)
  dst = edge_index[1].astype(jnp.int32)
  rid = rel_id.astype(jnp.int32)
  zacc = jnp.zeros((ACC_PAD, QD), jnp.float32)
  zden = jnp.zeros((ACC_PAD,), jnp.float32)
  w0 = jnp.stack([We0, Wn0, Wc0])
  w1 = jnp.stack([We1, Wn1, Wc1])
  ent1 = _layer(ent_emb, rel_emb0, w0, src, dst, rid, zacc, zden)
  ent2 = _layer(ent1, rel_emb1, w1, src, dst, rid, zacc, zden)
  hq = _headq_kernel(ent2, pred_rel, h_id.astype(jnp.int32),
                     r_id.astype(jnp.int32))
  return _score(hq, ent2)


# speedup vs baseline: 5.6289x; 1.8134x over previous
import functools
import jax
import jax.numpy as jnp
from jax import lax
from jax.experimental import pallas as pl
from jax.experimental.pallas import tpu as pltpu, tpu_sc as plsc

N_NODES = 10000
N_RELROWS = 1000
E_EDGES = 160000
DIM = 256
BATCH = 512

NC = 2
NS = 16
NWORK = NC * NS
EB = 128
NBLK = E_EDGES // EB
EB2 = 256
NBLK2 = E_EDGES // EB2
QD = 32
NQ = DIM // QD
NROUND = NQ // NC
ACC_ROWS = 3 * N_NODES
ACC_PAD = 30720
STRIPE = ACC_PAD // NS
RB = 128
NWB = STRIPE // RB

_mesh = plsc.VectorSubcoreMesh(
    core_axis_name="c", subcore_axis_name="s", num_cores=NC, num_subcores=NS)
_sc_params = pltpu.CompilerParams(
    use_tc_tiling_on_sc=False, needs_layout_passes=False)


def _iota16():
  return lax.iota(jnp.int32, 16)


def _full16(v):
  return jnp.full((16,), v, jnp.int32)


def _scores_body(ent_hbm, rel_hbm, src_hbm, dst_hbm, rid_hbm, zden_hbm,
                 ex_hbm, den_hbm,
                 srcv, dstv, ridv, u_rows, v_rows, e_rows, pbuf, exd, sidx,
                 den_sh, sem, sem2, sem3):
  c = lax.axis_index("c")
  s = lax.axis_index("s")
  w = s * NC + c
  nb = (NBLK - w + NWORK - 1) // NWORK

  @pl.when(s == 0)
  def _init():
    pltpu.sync_copy(zden_hbm, den_sh)

  plsc.subcore_barrier()

  def blk_body(t, carry):
    blk = w + t * NWORK
    base = blk * EB
    d1 = pltpu.async_copy(src_hbm.at[pl.ds(base, EB)], srcv, sem)
    d2 = pltpu.async_copy(dst_hbm.at[pl.ds(base, EB)], dstv, sem2)
    d3 = pltpu.async_copy(rid_hbm.at[pl.ds(base, EB)], ridv, sem3)
    d1.wait()
    d2.wait()
    d3.wait()
    g1 = pltpu.async_copy(ent_hbm.at[srcv], u_rows, sem)
    g2 = pltpu.async_copy(ent_hbm.at[dstv], v_rows, sem2)
    g3 = pltpu.async_copy(rel_hbm.at[ridv], e_rows, sem3)
    g1.wait()
    g2.wait()
    g3.wait()

    def edge(i, carry2):
      z = jnp.zeros((16,), jnp.float32)
      a1 = z
      a2 = z
      a3 = z
      for cc in range(DIM // 16):
        sl = pl.ds(cc * 16, 16)
        u = u_rows[i, sl]
        v = v_rows[i, sl]
        e = e_rows[i, sl]
        ev = e * v
        a1 = a1 + ev
        a2 = a2 + u * v
        a3 = a3 + u * ev
      pbuf[i] = plsc.cumsum(a1)
      pbuf[EB + i] = plsc.cumsum(a2)
      pbuf[2 * EB + i] = plsc.cumsum(a3)
      return carry2

    lax.fori_loop(0, EB, edge, 0)
    f15 = _full16(15)
    for g in range(EB // 16):
      rows16 = _iota16() + g * 16
      dv = dstv[pl.ds(g * 16, 16)]
      for j in range(3):
        ssum = plsc.load_gather(pbuf, [j * EB + rows16, f15])
        exd[pl.ds(j * EB + g * 16, 16)] = jnp.exp(ssum)
        sidx[pl.ds(j * EB + g * 16, 16)] = dv + j * N_NODES
    pltpu.sync_copy(exd, den_sh.at[sidx], add=True)
    for j in range(3):
      pltpu.sync_copy(exd.at[pl.ds(j * EB, EB)], ex_hbm.at[j, pl.ds(base, EB)])
    return carry

  lax.fori_loop(0, nb, blk_body, 0)
  plsc.subcore_barrier()
  pltpu.sync_copy(den_sh.at[pl.ds(s * STRIPE, STRIPE)],
                  den_hbm.at[c, pl.ds(s * STRIPE, STRIPE)])


_scores_kernel = pl.kernel(
    _scores_body,
    out_type=(jax.ShapeDtypeStruct((3, E_EDGES), jnp.float32),
              jax.ShapeDtypeStruct((NC, ACC_PAD), jnp.float32)),
    mesh=_mesh,
    scratch_types=[
        pltpu.VMEM((EB,), jnp.int32),
        pltpu.VMEM((EB,), jnp.int32),
        pltpu.VMEM((EB,), jnp.int32),
        pltpu.VMEM((EB, DIM), jnp.float32),
        pltpu.VMEM((EB, DIM), jnp.float32),
        pltpu.VMEM((EB, DIM), jnp.float32),
        pltpu.VMEM((3 * EB, 16), jnp.float32),
        pltpu.VMEM((3 * EB,), jnp.float32),
        pltpu.VMEM((3 * EB,), jnp.int32),
        pltpu.VMEM_SHARED((ACC_PAD,), jnp.float32),
        pltpu.SemaphoreType.DMA,
        pltpu.SemaphoreType.DMA,
        pltpu.SemaphoreType.DMA,
    ],
    compiler_params=_sc_params,
)


def _agg_body(entq_hbm, relq_hbm, src_hbm, dst_hbm, rid_hbm, ex_hbm,
              zacc_hbm, out_hbm,
              srcv, dstv, ridv, giu, gie, sidx, exd, u_q, e_q, msg,
              acc_sh, sem, sem2, sem3, sem4, sem5):
  c = lax.axis_index("c")
  s = lax.axis_index("s")
  stripe = pl.ds(s * STRIPE, STRIPE)
  pltpu.sync_copy(zacc_hbm.at[stripe], acc_sh.at[stripe])
  nb = (NBLK2 - s + NS - 1) // NS

  for rnd in range(NROUND):
    plsc.subcore_barrier()
    q = rnd * NC + c
    qent = q * N_NODES
    qrel = q * N_RELROWS

    def blk_body(t, carry):
      blk = s + t * NS
      base = blk * EB2
      d1 = pltpu.async_copy(src_hbm.at[pl.ds(base, EB2)], srcv, sem)
      d2 = pltpu.async_copy(dst_hbm.at[pl.ds(base, EB2)], dstv, sem2)
      d3 = pltpu.async_copy(rid_hbm.at[pl.ds(base, EB2)], ridv, sem3)
      d4 = [pltpu.async_copy(ex_hbm.at[j, pl.ds(base, EB2)],
                             exd.at[pl.ds(j * EB2, EB2)], sem4)
            for j in range(3)]
      d1.wait()
      d2.wait()
      d3.wait()
      for g in range(EB2 // 16):
        sl = pl.ds(g * 16, 16)
        giu[sl] = srcv[sl] + qent
        gie[sl] = ridv[sl] + qrel
      g1 = pltpu.async_copy(entq_hbm.at[giu], u_q, sem)
      g2 = pltpu.async_copy(relq_hbm.at[gie], e_q, sem2)

      @pl.when(t > 0)
      def _drain():
        pltpu.make_async_copy(msg, acc_sh.at[sidx], sem5).wait()

      for g in range(EB2 // 16):
        sl = pl.ds(g * 16, 16)
        dv = dstv[sl]
        sidx[sl] = dv
        sidx[pl.ds(EB2 + g * 16, 16)] = dv + N_NODES
        sidx[pl.ds(2 * EB2 + g * 16, 16)] = dv + 2 * N_NODES
      for d in d4:
        d.wait()
      g1.wait()
      g2.wait()

      def edge(i2, carry2):
        for k in range(2):
          i = i2 * 2 + k
          e1 = plsc.load_gather(exd, [_full16(i)])
          e2 = plsc.load_gather(exd, [_full16(EB2 + i)])
          e3 = plsc.load_gather(exd, [_full16(2 * EB2 + i)])
          for cc in range(QD // 16):
            sl = pl.ds(cc * 16, 16)
            u = u_q[i, sl]
            e = e_q[i, sl]
            msg[i, sl] = e * e1
            msg[EB2 + i, sl] = u * e2
            msg[2 * EB2 + i, sl] = u * e * e3
        return carry2

      lax.fori_loop(0, EB2 // 2, edge, 0)
      pltpu.async_copy(msg, acc_sh.at[sidx], sem5, add=True)
      return carry

    lax.fori_loop(0, nb, blk_body, 0)

    @pl.when(nb > 0)
    def _final_drain():
      pltpu.make_async_copy(msg, acc_sh.at[sidx], sem5).wait()

    plsc.subcore_barrier()
    pltpu.sync_copy(acc_sh.at[stripe], out_hbm.at[rnd, c, stripe])
    if rnd + 1 < NROUND:
      pltpu.sync_copy(zacc_hbm.at[stripe], acc_sh.at[stripe])


def _make_agg_kernel():
  return pl.kernel(
      _agg_body,
      out_type=jax.ShapeDtypeStruct((NROUND, NC, ACC_PAD, QD), jnp.float32),
      mesh=_mesh,
      scratch_types=[
          pltpu.VMEM((EB2,), jnp.int32),
          pltpu.VMEM((EB2,), jnp.int32),
          pltpu.VMEM((EB2,), jnp.int32),
          pltpu.VMEM((EB2,), jnp.int32),
          pltpu.VMEM((EB2,), jnp.int32),
          pltpu.VMEM((3 * EB2,), jnp.int32),
          pltpu.VMEM((3 * EB2,), jnp.float32),
          pltpu.VMEM((EB2, QD), jnp.float32),
          pltpu.VMEM((EB2, QD), jnp.float32),
          pltpu.VMEM((3 * EB2, QD), jnp.float32),
          pltpu.VMEM_SHARED((ACC_PAD, QD), jnp.float32),
          pltpu.SemaphoreType.DMA,
          pltpu.SemaphoreType.DMA,
          pltpu.SemaphoreType.DMA,
          pltpu.SemaphoreType.DMA,
          pltpu.SemaphoreType.DMA,
      ],
      compiler_params=_sc_params,
  )


_agg_kernel = _make_agg_kernel()


def _combine_body(ent_ref, m_ref, den_ref, w_ref, out_ref):
  den = den_ref[...]
  den = den[:, :, 0] + den[:, :, 1] + 1e-16
  acc = ent_ref[...]
  for b in range(3):
    mb = m_ref[b] * (1.0 / den[:, b])[:, None]
    acc = acc + jnp.tanh(
        jnp.dot(mb, w_ref[b], preferred_element_type=jnp.float32))
  out_ref[...] = acc


_ROWBLK = 1000


def _combine(ent, m, den_t, w):
  return pl.pallas_call(
      _combine_body,
      grid=(N_NODES // _ROWBLK,),
      in_specs=[
          pl.BlockSpec((_ROWBLK, DIM), lambda i: (i, 0)),
          pl.BlockSpec((3, _ROWBLK, DIM), lambda i: (0, i, 0)),
          pl.BlockSpec((_ROWBLK, 3, 2), lambda i: (i, 0, 0)),
          pl.BlockSpec((3, DIM, DIM), lambda i: (0, 0, 0)),
      ],
      out_specs=pl.BlockSpec((_ROWBLK, DIM), lambda i: (i, 0)),
      out_shape=jax.ShapeDtypeStruct((N_NODES, DIM), jnp.float32),
  )(ent, m, den_t, w)


def _headq_body(ent_hbm, prel_hbm, h_hbm, r_hbm, out_hbm,
                hv, rv, hrows, rrows, obuf, sem):
  c = lax.axis_index("c")
  s = lax.axis_index("s")
  w = s * NC + c
  base = w * 16
  pltpu.sync_copy(h_hbm.at[pl.ds(base, 16)], hv)
  pltpu.sync_copy(r_hbm.at[pl.ds(base, 16)], rv)
  pltpu.async_copy(ent_hbm.at[hv], hrows, sem).wait()
  pltpu.async_copy(prel_hbm.at[rv], rrows, sem).wait()

  def row(i, carry):
    for cc in range(DIM // 16):
      sl = pl.ds(cc * 16, 16)
      obuf[i, sl] = hrows[i, sl] * rrows[i, sl]
    return carry

  lax.fori_loop(0, 16, row, 0)
  pltpu.sync_copy(obuf, out_hbm.at[pl.ds(base, 16)])


_headq_kernel = pl.kernel(
    _headq_body,
    out_type=jax.ShapeDtypeStruct((BATCH, DIM), jnp.float32),
    mesh=_mesh,
    scratch_types=[
        pltpu.VMEM((16,), jnp.int32),
        pltpu.VMEM((16,), jnp.int32),
        pltpu.VMEM((16, DIM), jnp.float32),
        pltpu.VMEM((16, DIM), jnp.float32),
        pltpu.VMEM((16, DIM), jnp.float32),
        pltpu.SemaphoreType.DMA,
    ],
    compiler_params=_sc_params,
)


def _score_body(hq_ref, ent_ref, out_ref):
  out_ref[...] = lax.dot_general(
      hq_ref[...], ent_ref[...],
      dimension_numbers=(((1,), (1,)), ((), ())),
      preferred_element_type=jnp.float32)


def _score(hq, ent):
  return pl.pallas_call(
      _score_body,
      out_shape=jax.ShapeDtypeStruct((BATCH, N_NODES), jnp.float32),
  )(hq, ent)


def _slices(table):
  r = table.shape[0]
  return jnp.transpose(table.reshape(r, NQ, QD), (1, 0, 2)).reshape(NQ * r, QD)


def _layer(ent, rel, wstack, src, dst, rid, zacc, zden):
  ex, den = _scores_kernel(ent, rel, src, dst, rid, zden)
  entq = _slices(ent)
  relq = _slices(rel)
  outs = _agg_kernel(entq, relq, src, dst, rid, ex, zacc)
  m = jnp.concatenate([outs[r, c] for r in range(NROUND)
                       for c in range(NC)], axis=-1)
  m = m[:ACC_ROWS].reshape(3, N_NODES, DIM)
  den_t = jnp.transpose(den[:, :ACC_ROWS].reshape(NC, 3, N_NODES), (2, 1, 0))
  return _combine(ent, m, den_t, wstack)


def kernel(ent_emb, rel_emb0, rel_emb1, We0, Wn0, Wc0, We1, Wn1, Wc1,
           pred_rel, h_id, r_id, edge_index, rel_id):
  src = edge_index[0].astype(jnp.int32)
  dst = edge_index[1].astype(jnp.int32)
  rid = rel_id.astype(jnp.int32)
  zacc = jnp.zeros((ACC_PAD, QD), jnp.float32)
  zden = jnp.zeros((ACC_PAD,), jnp.float32)
  w0 = jnp.stack([We0, Wn0, Wc0])
  w1 = jnp.stack([We1, Wn1, Wc1])
  ent1 = _layer(ent_emb, rel_emb0, w0, src, dst, rid, zacc, zden)
  ent2 = _layer(ent1, rel_emb1, w1, src, dst, rid, zacc, zden)
  hq = _headq_kernel(ent2, pred_rel, h_id.astype(jnp.int32),
                     r_id.astype(jnp.int32))
  return _score(hq, ent2)
